# Initial kernel scaffold; baseline (speedup 1.0000x reference)
import jax
import jax.numpy as jnp
from jax.experimental import pallas as pl


def kernel(x, edge_index, edge_attr, batch, demographics, emb, W_in, b_in, W_rel, b_rel, W_root, Wd0, bd0, Wd1, bd1, Wds1, bds1, Wds2, bds2, gn_w, gn_b, gn_ms, Wg1, bg1, Wg2, bg2, Wc, bc):
    raise NotImplementedError("write your pallas kernel here")


# speedup vs baseline: 3.1800x; 3.1800x over previous
import functools

import jax
import jax.numpy as jnp
from jax import lax
from jax.experimental import pallas as pl
from jax.experimental.pallas import tpu as pltpu
from jax.experimental.pallas import tpu_sc as plsc

_N = 50000
_E = 800000
_NG = 16
_H = 64
_L = 3

_G = 128
_NCHUNK = _E // _G
_NSUB = 16
_CHUNK_ITERS = -(-_NCHUNK // _NSUB)
_HALF = _N // 2
_SPROWS = 25600
_ZR = 160
_RPS = _SPROWS // _NSUB


def _message_pass(h, src, dst, ew):
    mesh = plsc.VectorSubcoreMesh(core_axis_name="c", subcore_axis_name="s")

    @functools.partial(
        pl.kernel,
        mesh=mesh,
        compiler_params=pltpu.CompilerParams(use_tc_tiling_on_sc=False),
        out_type=jax.ShapeDtypeStruct((2 * _SPROWS, _H), jnp.float32),
        scratch_types=[
            pltpu.VMEM((_G,), jnp.int32),
            pltpu.VMEM((_G,), jnp.int32),
            pltpu.VMEM((_G,), jnp.float32),
            pltpu.VMEM((_G, _H), jnp.float32),
            pltpu.VMEM((_ZR, _H), jnp.float32),
            pltpu.VMEM_SHARED((_SPROWS, _H), jnp.float32),
            pltpu.SemaphoreType.DMA,
        ],
    )
    def k(h_hbm, src_hbm, dst_hbm, ew_hbm, out_hbm,
          sidx, didx, ewv, rows, zv, aggsh, sem):
        c = lax.axis_index("c")
        s = lax.axis_index("s")
        base_node = c * _HALF
        zero16 = jnp.zeros((16,), jnp.float32)
        lane_iota = lax.iota(jnp.int32, 16)

        for r in range(_ZR):
            for q in range(_H // 16):
                zv[r, pl.ds(q * 16, 16)] = zero16
        for t in range(_RPS // _ZR):
            pltpu.sync_copy(zv, aggsh.at[pl.ds(s * _RPS + t * _ZR, _ZR)])
        plsc.subcore_barrier()

        def body(j, carry):
            cid = j * _NSUB + s

            @pl.when(cid < _NCHUNK)
            def _():
                ebase = cid * _G
                pltpu.sync_copy(src_hbm.at[pl.ds(ebase, _G)], sidx)
                pltpu.sync_copy(dst_hbm.at[pl.ds(ebase, _G)], didx)
                pltpu.sync_copy(ew_hbm.at[pl.ds(ebase, _G)], ewv)
                pltpu.async_copy(h_hbm.at[sidx], rows, sem).wait()
                for q in range(_G // 16):
                    dv = didx[pl.ds(q * 16, 16)]
                    lv = dv - base_node
                    ok = (lv >= 0) & (lv < _HALF)
                    didx[pl.ds(q * 16, 16)] = jnp.where(ok, lv, _HALF)
                for q in range(_G // 16):
                    evec = ewv[pl.ds(q * 16, 16)]
                    for lane in range(16):
                        g = q * 16 + lane
                        wv = lax.broadcast_in_dim(evec[lane], (16,), ())
                        for qq in range(_H // 16):
                            sl = pl.ds(qq * 16, 16)
                            rows[g, sl] = rows[g, sl] * wv
                pltpu.sync_copy(rows, aggsh.at[didx], add=True)
            return carry

        lax.fori_loop(0, _CHUNK_ITERS, body, 0)
        plsc.subcore_barrier()
        pltpu.sync_copy(
            aggsh.at[pl.ds(s * _RPS, _RPS)],
            out_hbm.at[pl.ds(c * _SPROWS + s * _RPS, _RPS)])

    padded = k(h, src, dst, ew)
    return jnp.concatenate(
        [padded[:_HALF], padded[_SPROWS:_SPROWS + _HALF]], axis=0)


def _leaky_relu(v):
    return jnp.where(v >= 0, v, 0.01 * v)


def _elu(v):
    return jnp.where(v > 0, v, jnp.expm1(v))


def kernel(x, edge_index, edge_attr, batch, demographics, emb, W_in, b_in,
           W_rel, b_rel, W_root, Wd0, bd0, Wd1, bd1,
           Wds1, bds1, Wds2, bds2, gn_w, gn_b, gn_ms,
           Wg1, bg1, Wg2, bg2, Wc, bc):
    src = edge_index[0].astype(jnp.int32)
    dst = edge_index[1].astype(jnp.int32)
    ew = edge_attr[:, 0].astype(jnp.float32)

    h = emb[x]
    h = h @ W_in.T + b_in

    onehot = (batch[:, None] == jnp.arange(_NG)[None, :]).astype(jnp.float32)
    counts = jnp.sum(onehot, axis=0)

    demo = demographics
    demo_params = [(Wd0, bd0), (Wd1, bd1)]
    for i in range(_L):
        agg = _message_pass(h, src, dst, ew)
        h = agg @ W_rel[i].T + b_rel[i] + h @ W_root[i].T
        if i < _L - 1:
            Wd, bd = demo_params[i]
            demo = demo @ Wd.T + bd
            demo_exp = onehot @ demo
            comb = jnp.concatenate([h, demo_exp], axis=1)
            t = _leaky_relu(comb @ Wds1[i].T + bds1[i])
            h = t @ Wds2[i].T + bds2[i]
            mean = (onehot.T @ h) / counts[:, None]
            centered = h - (onehot @ mean) * gn_ms[i]
            var = (onehot.T @ (centered * centered)) / counts[:, None]
            h = gn_w[i] * centered / jnp.sqrt(onehot @ var + 1e-5) + gn_b[i]
            h = _elu(h)

    gate = _elu(h @ Wg1.T + bg1) @ Wg2.T + bg2
    g = gate[:, 0]
    gmax = jnp.max(jnp.where(onehot > 0, g[:, None], -jnp.inf), axis=0)
    eg = jnp.exp(g - onehot @ gmax)
    denom = onehot.T @ eg
    alpha = eg / (onehot @ denom)
    pooled = onehot.T @ (alpha[:, None] * h)
    return pooled @ Wc.T + bc
